# baseline (speedup 1.0000x reference)
import functools

import jax
import jax.numpy as jnp
from jax import lax
from jax.experimental import pallas as pl
from jax.experimental.pallas import tpu as pltpu
from jax.experimental.pallas import tpu_sc as plsc

NC = 2
NS = 16
NW = NC * NS
C = 128
NBUF = 3


def _log_1to2(x):
    s = (x - 1.0) / (x + 1.0)
    s2 = s * s
    p = 1.0 / 9.0
    p = p * s2 + 1.0 / 7.0
    p = p * s2 + 1.0 / 5.0
    p = p * s2 + 1.0 / 3.0
    p = p * s2 + 1.0
    return (2.0 * s) * p


def _sqrt(x):
    i = lax.bitcast_convert_type(x, jnp.int32)
    y = lax.bitcast_convert_type(jnp.int32(0x5F3759DF) - (i >> 1), jnp.float32)
    for _ in range(3):
        y = y * (1.5 - 0.5 * x * y * y)
    return x * y


def _make_tc_transpose(V, D):
    VP = -(-V // 128) * 128
    grid = VP // 128

    return pl.pallas_call(
        _tc_transpose_body,
        grid=(grid,),
        in_specs=[pl.BlockSpec((D, 128), lambda b: (0, b))],
        out_specs=pl.BlockSpec((128, 2 * D), lambda b: (b, 0)),
        out_shape=jax.ShapeDtypeStruct((VP, 2 * D), jnp.float32),
    )


def _tc_transpose_body(wt_ref, out_ref):
    x = wt_ref[...]
    out_ref[:, 0:x.shape[0]] = x.T


def kernel(pos_idxs, ys, neg_idxs, num_neg, W):
    B = pos_idxs.shape[1]
    NT = neg_idxs.shape[1]
    V, D = W.shape
    TOT = B + NT
    ppw = TOT // NW
    nch = ppw // C
    assert ppw % C == 0 and nch % NBUF == 0 and D == 64
    scale = 1.0 / float(TOT)

    mesh = plsc.VectorSubcoreMesh(core_axis_name="c", subcore_axis_name="s")

    @functools.partial(
        pl.kernel,
        mesh=mesh,
        compiler_params=pltpu.CompilerParams(
            needs_layout_passes=False, use_tc_tiling_on_sc=True),
        out_type=jax.ShapeDtypeStruct((NW, 16), jnp.float32),
        scratch_types=(
            [pltpu.VMEM((ppw,), jnp.int32) for _ in range(2)]
            + [pltpu.VMEM((ppw,), jnp.float32)]
            + [pltpu.VMEM((C,), jnp.float32)]
            + [pltpu.VMEM((C, 2 * D), jnp.float32) for _ in range(2 * NBUF)]
            + [pltpu.VMEM((16,), jnp.float32)]
            + [pltpu.SemaphoreType.DMA for _ in range(2 * NBUF)]
        ),
    )
    def sc_loss(i0_h, i1_h, yse_h, w2_h, out_h, *refs):
        rawu, rawv, ysa, dots = refs[0:4]
        urows = refs[4:4 + NBUF]
        vrows = refs[4 + NBUF:4 + 2 * NBUF]
        accv = refs[4 + 2 * NBUF]
        semu = refs[5 + 2 * NBUF:5 + 2 * NBUF + NBUF]
        semv = refs[5 + 2 * NBUF + NBUF:5 + 2 * NBUF + 2 * NBUF]

        wid = lax.axis_index("s") * NC + lax.axis_index("c")
        tbase = wid * ppw
        lanes = lax.iota(jnp.int32, 16)

        pltpu.sync_copy(i0_h.at[pl.ds(tbase, ppw)], rawu)
        pltpu.sync_copy(i1_h.at[pl.ds(tbase, ppw)], rawv)
        pltpu.sync_copy(yse_h.at[pl.ds(tbase, ppw)], ysa)

        def fire(c, r):
            pltpu.async_copy(
                w2_h.at[rawu.at[pl.ds(c * C, C)]], urows[r], semu[r])
            pltpu.async_copy(
                w2_h.at[rawv.at[pl.ds(c * C, C)]], vrows[r], semv[r])

        def wait(c, r):
            pltpu.make_async_copy(
                w2_h.at[rawu.at[pl.ds(c * C, C)]], urows[r], semu[r]).wait()
            pltpu.make_async_copy(
                w2_h.at[rawv.at[pl.ds(c * C, C)]], vrows[r], semv[r]).wait()

        def compute(c, r, acc):
            def pair16(p16, _):
                dv = jnp.zeros((16,), jnp.float32)
                for q in range(16):
                    p = p16 * 16 + q
                    s = jnp.zeros((16,), jnp.float32)
                    for t in range(D // 16):
                        a = urows[r][p, pl.ds(t * 16, 16)]
                        b = vrows[r][p, pl.ds(t * 16, 16)]
                        s = s + a * b
                    dv = jnp.where(lanes == q, jnp.sum(s), dv)
                dots[pl.ds(p16 * 16, 16)] = dv
                return 0

            lax.fori_loop(0, C // 16, pair16, 0)

            def group(g, acc):
                avec = c * C + g * 16 + lanes
                dot = plsc.load_gather(dots, [g * 16 + lanes])
                z = jnp.where(tbase + avec < B, -dot, dot)
                t = jnp.exp(-jnp.abs(z))
                sp = jnp.maximum(z, 0.0) + _log_1to2(1.0 + t)
                yv = plsc.load_gather(ysa, [avec])
                alpha = _log_1to2(1.0 + _sqrt(yv)) + 1.0
                return acc + alpha * sp

            return lax.fori_loop(0, C // 16, group, acc)

        fire(0, 0)
        fire(1, 1)

        def step(k, acc):
            for r in range(NBUF):
                c = k * NBUF + r
                fire(c + 2, (r + 2) % NBUF)
                wait(c, r)
                acc = compute(c, r, acc)
            return acc

        acc = lax.fori_loop(0, nch // NBUF - 1, step,
                            jnp.zeros((16,), jnp.float32))
        c = nch - 3
        fire(nch - 1, (nch - 1) % NBUF)
        wait(c, c % NBUF)
        acc = compute(c, c % NBUF, acc)
        for c in range(nch - 2, nch):
            wait(c, c % NBUF)
            acc = compute(c, c % NBUF, acc)

        accv[...] = acc * scale
        pltpu.sync_copy(accv, out_h.at[wid])

    i0 = jnp.concatenate([pos_idxs[0], neg_idxs[0]])
    i1 = jnp.concatenate([pos_idxs[1], neg_idxs[1]])
    yse = jnp.concatenate([ys, jnp.zeros((NT,), jnp.float32)])
    w2 = _make_tc_transpose(V, D)(W.T)
    partials = sc_loss(i0, i1, yse, w2)
    return jnp.sum(partials)


# speedup vs baseline: 7.8250x; 7.8250x over previous
import functools

import jax
import jax.numpy as jnp
from jax import lax
from jax.experimental import pallas as pl
from jax.experimental.pallas import tpu as pltpu
from jax.experimental.pallas import tpu_sc as plsc

NC = 2
NS = 16
NW = NC * NS
C = 128
NBUF = 3


def _log_1to2(x):
    s = (x - 1.0) / (x + 1.0)
    s2 = s * s
    p = 1.0 / 9.0
    p = p * s2 + 1.0 / 7.0
    p = p * s2 + 1.0 / 5.0
    p = p * s2 + 1.0 / 3.0
    p = p * s2 + 1.0
    return (2.0 * s) * p


def _sqrt(x):
    i = lax.bitcast_convert_type(x, jnp.int32)
    y = lax.bitcast_convert_type(jnp.int32(0x5F3759DF) - (i >> 1), jnp.float32)
    for _ in range(3):
        y = y * (1.5 - 0.5 * x * y * y)
    return x * y


def _make_tc_transpose(V, D):
    BW = 2048
    VP = -(-V // BW) * BW
    grid = VP // BW

    return pl.pallas_call(
        _tc_transpose_body,
        grid=(grid,),
        in_specs=[pl.BlockSpec((D, BW), lambda b: (0, b))],
        out_specs=pl.BlockSpec((BW, 2 * D), lambda b: (b, 0)),
        out_shape=jax.ShapeDtypeStruct((VP, 2 * D), jnp.float32),
    )


def _tc_transpose_body(wt_ref, out_ref):
    x = wt_ref[...]
    out_ref[:, 0:x.shape[0]] = x.T


def kernel(pos_idxs, ys, neg_idxs, num_neg, W):
    B = pos_idxs.shape[1]
    NT = neg_idxs.shape[1]
    V, D = W.shape
    TOT = B + NT
    ppw = TOT // NW
    nch = ppw // C
    assert ppw % C == 0 and nch % NBUF == 0 and D == 64
    scale = 1.0 / float(TOT)

    mesh = plsc.VectorSubcoreMesh(core_axis_name="c", subcore_axis_name="s")

    @functools.partial(
        pl.kernel,
        mesh=mesh,
        compiler_params=pltpu.CompilerParams(
            needs_layout_passes=False, use_tc_tiling_on_sc=True),
        out_type=jax.ShapeDtypeStruct((NW, 16), jnp.float32),
        scratch_types=(
            [pltpu.VMEM((ppw,), jnp.int32) for _ in range(2)]
            + [pltpu.VMEM((ppw,), jnp.float32)]
            + [pltpu.VMEM((C,), jnp.float32)]
            + [pltpu.VMEM((C, 2 * D), jnp.float32) for _ in range(2 * NBUF)]
            + [pltpu.VMEM((16,), jnp.float32)]
            + [pltpu.SemaphoreType.DMA for _ in range(2 * NBUF)]
        ),
    )
    def sc_loss(i0_h, i1_h, yse_h, w2_h, out_h, *refs):
        rawu, rawv, ysa, dots = refs[0:4]
        urows = refs[4:4 + NBUF]
        vrows = refs[4 + NBUF:4 + 2 * NBUF]
        accv = refs[4 + 2 * NBUF]
        semu = refs[5 + 2 * NBUF:5 + 2 * NBUF + NBUF]
        semv = refs[5 + 2 * NBUF + NBUF:5 + 2 * NBUF + 2 * NBUF]

        wid = lax.axis_index("s") * NC + lax.axis_index("c")
        tbase = wid * ppw
        lanes = lax.iota(jnp.int32, 16)

        pltpu.sync_copy(i0_h.at[pl.ds(tbase, ppw)], rawu)
        pltpu.sync_copy(i1_h.at[pl.ds(tbase, ppw)], rawv)
        pltpu.sync_copy(yse_h.at[pl.ds(tbase, ppw)], ysa)

        def fire(c, r):
            pltpu.async_copy(
                w2_h.at[rawu.at[pl.ds(c * C, C)]], urows[r], semu[r])
            pltpu.async_copy(
                w2_h.at[rawv.at[pl.ds(c * C, C)]], vrows[r], semv[r])

        def wait(c, r):
            pltpu.make_async_copy(
                w2_h.at[rawu.at[pl.ds(c * C, C)]], urows[r], semu[r]).wait()
            pltpu.make_async_copy(
                w2_h.at[rawv.at[pl.ds(c * C, C)]], vrows[r], semv[r]).wait()

        def compute(c, r, acc):
            def pair16(p16, _):
                dv = jnp.zeros((16,), jnp.float32)
                for q in range(16):
                    p = p16 * 16 + q
                    s = jnp.zeros((16,), jnp.float32)
                    for t in range(D // 16):
                        a = urows[r][p, pl.ds(t * 16, 16)]
                        b = vrows[r][p, pl.ds(t * 16, 16)]
                        s = s + a * b
                    dv = jnp.where(lanes == q, jnp.sum(s), dv)
                dots[pl.ds(p16 * 16, 16)] = dv
                return 0

            lax.fori_loop(0, C // 16, pair16, 0)

            def group(g, acc):
                avec = c * C + g * 16 + lanes
                dot = plsc.load_gather(dots, [g * 16 + lanes])
                z = jnp.where(tbase + avec < B, -dot, dot)
                t = jnp.exp(-jnp.abs(z))
                sp = jnp.maximum(z, 0.0) + _log_1to2(1.0 + t)
                yv = plsc.load_gather(ysa, [avec])
                alpha = _log_1to2(1.0 + _sqrt(yv)) + 1.0
                return acc + alpha * sp

            return lax.fori_loop(0, C // 16, group, acc)

        fire(0, 0)
        fire(1, 1)

        def step(k, acc):
            for r in range(NBUF):
                c = k * NBUF + r
                fire(c + 2, (r + 2) % NBUF)
                wait(c, r)
                acc = compute(c, r, acc)
            return acc

        acc = lax.fori_loop(0, nch // NBUF - 1, step,
                            jnp.zeros((16,), jnp.float32))
        c = nch - 3
        fire(nch - 1, (nch - 1) % NBUF)
        wait(c, c % NBUF)
        acc = compute(c, c % NBUF, acc)
        for c in range(nch - 2, nch):
            wait(c, c % NBUF)
            acc = compute(c, c % NBUF, acc)

        accv[...] = acc * scale
        pltpu.sync_copy(accv, out_h.at[wid])

    i0 = jnp.concatenate([pos_idxs[0], neg_idxs[0]])
    i1 = jnp.concatenate([pos_idxs[1], neg_idxs[1]])
    yse = jnp.concatenate([ys, jnp.zeros((NT,), jnp.float32)])
    w2 = _make_tc_transpose(V, D)(W.T)
    partials = sc_loss(i0, i1, yse, w2)
    return jnp.sum(partials)


# speedup vs baseline: 10.2203x; 1.3061x over previous
import functools

import jax
import jax.numpy as jnp
from jax import lax
from jax.experimental import pallas as pl
from jax.experimental.pallas import tpu as pltpu
from jax.experimental.pallas import tpu_sc as plsc

NC = 2
NS = 16
NW = NC * NS
C = 128
NBUF = 3


def _log_1to2(x):
    s = (x - 1.0) / (x + 1.0)
    s2 = s * s
    p = 1.0 / 9.0
    p = p * s2 + 1.0 / 7.0
    p = p * s2 + 1.0 / 5.0
    p = p * s2 + 1.0 / 3.0
    p = p * s2 + 1.0
    return (2.0 * s) * p


def _sqrt(x):
    i = lax.bitcast_convert_type(x, jnp.int32)
    y = lax.bitcast_convert_type(jnp.int32(0x5F3759DF) - (i >> 1), jnp.float32)
    for _ in range(3):
        y = y * (1.5 - 0.5 * x * y * y)
    return x * y


def _make_tc_transpose(V, D):
    BW = 4096
    VP = -(-V // BW) * BW
    grid = VP // BW

    return pl.pallas_call(
        _tc_transpose_body,
        grid=(grid,),
        in_specs=[pl.BlockSpec((D, BW), lambda b: (0, b))],
        out_specs=pl.BlockSpec((BW, 2 * D), lambda b: (b, 0)),
        out_shape=jax.ShapeDtypeStruct((VP, 2 * D), jnp.float32),
    )


def _tc_transpose_body(wt_ref, out_ref):
    x = wt_ref[...]
    out_ref[:, 0:x.shape[0]] = x.T


def kernel(pos_idxs, ys, neg_idxs, num_neg, W):
    B = pos_idxs.shape[1]
    NT = neg_idxs.shape[1]
    V, D = W.shape
    TOT = B + NT
    ppw = TOT // NW
    nch = ppw // C
    assert ppw % C == 0 and nch % NBUF == 0 and D == 64
    scale = 1.0 / float(TOT)

    mesh = plsc.VectorSubcoreMesh(core_axis_name="c", subcore_axis_name="s")

    @functools.partial(
        pl.kernel,
        mesh=mesh,
        compiler_params=pltpu.CompilerParams(
            needs_layout_passes=False, use_tc_tiling_on_sc=True),
        out_type=jax.ShapeDtypeStruct((NW, 16), jnp.float32),
        scratch_types=(
            [pltpu.VMEM((ppw,), jnp.int32) for _ in range(2)]
            + [pltpu.VMEM((ppw,), jnp.float32)]
            + [pltpu.VMEM((C,), jnp.float32)]
            + [pltpu.VMEM((C, 2 * D), jnp.float32) for _ in range(2 * NBUF)]
            + [pltpu.VMEM((16,), jnp.float32)]
            + [pltpu.SemaphoreType.DMA for _ in range(2 * NBUF)]
        ),
    )
    def sc_loss(i0_h, i1_h, yse_h, w2_h, out_h, *refs):
        rawu, rawv, ysa, dots = refs[0:4]
        urows = refs[4:4 + NBUF]
        vrows = refs[4 + NBUF:4 + 2 * NBUF]
        accv = refs[4 + 2 * NBUF]
        semu = refs[5 + 2 * NBUF:5 + 2 * NBUF + NBUF]
        semv = refs[5 + 2 * NBUF + NBUF:5 + 2 * NBUF + 2 * NBUF]

        wid = lax.axis_index("s") * NC + lax.axis_index("c")
        tbase = wid * ppw
        lanes = lax.iota(jnp.int32, 16)

        pltpu.sync_copy(i0_h.at[pl.ds(tbase, ppw)], rawu)
        pltpu.sync_copy(i1_h.at[pl.ds(tbase, ppw)], rawv)
        pltpu.sync_copy(yse_h.at[pl.ds(tbase, ppw)], ysa)

        def fire(c, r):
            pltpu.async_copy(
                w2_h.at[rawu.at[pl.ds(c * C, C)]], urows[r], semu[r])
            pltpu.async_copy(
                w2_h.at[rawv.at[pl.ds(c * C, C)]], vrows[r], semv[r])

        def wait(c, r):
            pltpu.make_async_copy(
                w2_h.at[rawu.at[pl.ds(c * C, C)]], urows[r], semu[r]).wait()
            pltpu.make_async_copy(
                w2_h.at[rawv.at[pl.ds(c * C, C)]], vrows[r], semv[r]).wait()

        def compute(c, r, acc):
            def pair16(p16, _):
                dv = jnp.zeros((16,), jnp.float32)
                for q in range(16):
                    p = p16 * 16 + q
                    s = jnp.zeros((16,), jnp.float32)
                    for t in range(D // 16):
                        a = urows[r][p, pl.ds(t * 16, 16)]
                        b = vrows[r][p, pl.ds(t * 16, 16)]
                        s = s + a * b
                    dv = jnp.where(lanes == q, jnp.sum(s), dv)
                dots[pl.ds(p16 * 16, 16)] = dv
                return 0

            lax.fori_loop(0, C // 16, pair16, 0)

            def group(g, acc):
                avec = c * C + g * 16 + lanes
                dot = plsc.load_gather(dots, [g * 16 + lanes])
                z = jnp.where(tbase + avec < B, -dot, dot)
                t = jnp.exp(-jnp.abs(z))
                sp = jnp.maximum(z, 0.0) + _log_1to2(1.0 + t)
                yv = plsc.load_gather(ysa, [avec])
                alpha = _log_1to2(1.0 + _sqrt(yv)) + 1.0
                return acc + alpha * sp

            return lax.fori_loop(0, C // 16, group, acc)

        fire(0, 0)
        fire(1, 1)

        def step(k, acc):
            for r in range(NBUF):
                c = k * NBUF + r
                fire(c + 2, (r + 2) % NBUF)
                wait(c, r)
                acc = compute(c, r, acc)
            return acc

        acc = lax.fori_loop(0, nch // NBUF - 1, step,
                            jnp.zeros((16,), jnp.float32))
        c = nch - 3
        fire(nch - 1, (nch - 1) % NBUF)
        wait(c, c % NBUF)
        acc = compute(c, c % NBUF, acc)
        for c in range(nch - 2, nch):
            wait(c, c % NBUF)
            acc = compute(c, c % NBUF, acc)

        accv[...] = acc * scale
        pltpu.sync_copy(accv, out_h.at[wid])

    i0 = jnp.concatenate([pos_idxs[0], neg_idxs[0]])
    i1 = jnp.concatenate([pos_idxs[1], neg_idxs[1]])
    yse = jnp.concatenate([ys, jnp.zeros((NT,), jnp.float32)])
    w2 = _make_tc_transpose(V, D)(W.T)
    partials = sc_loss(i0, i1, yse, w2)
    return jnp.sum(partials)


# speedup vs baseline: 12.3424x; 1.2076x over previous
import functools

import jax
import jax.numpy as jnp
from jax import lax
from jax.experimental import pallas as pl
from jax.experimental.pallas import tpu as pltpu
from jax.experimental.pallas import tpu_sc as plsc

NC = 2
NS = 16
NW = NC * NS
C = 128
NBUF = 3


def _log_1to2(x):
    s = (x - 1.0) / (x + 1.0)
    s2 = s * s
    p = 1.0 / 9.0
    p = p * s2 + 1.0 / 7.0
    p = p * s2 + 1.0 / 5.0
    p = p * s2 + 1.0 / 3.0
    p = p * s2 + 1.0
    return (2.0 * s) * p


def _sqrt(x):
    i = lax.bitcast_convert_type(x, jnp.int32)
    y = lax.bitcast_convert_type(jnp.int32(0x5F3759DF) - (i >> 1), jnp.float32)
    for _ in range(3):
        y = y * (1.5 - 0.5 * x * y * y)
    return x * y


def _make_tc_transpose(V, D):
    BW = 8192
    VP = -(-V // BW) * BW
    grid = VP // BW

    return pl.pallas_call(
        _tc_transpose_body,
        grid=(grid,),
        in_specs=[pl.BlockSpec((D, BW), lambda b: (0, b))],
        out_specs=pl.BlockSpec((BW, 2 * D), lambda b: (b, 0)),
        out_shape=jax.ShapeDtypeStruct((VP, 2 * D), jnp.float32),
    )


def _tc_transpose_body(wt_ref, out_ref):
    x = wt_ref[...]
    out_ref[:, 0:x.shape[0]] = x.T


def kernel(pos_idxs, ys, neg_idxs, num_neg, W):
    B = pos_idxs.shape[1]
    NT = neg_idxs.shape[1]
    V, D = W.shape
    TOT = B + NT
    ppw = TOT // NW
    nch = ppw // C
    assert ppw % C == 0 and nch % NBUF == 0 and D == 64
    scale = 1.0 / float(TOT)

    mesh = plsc.VectorSubcoreMesh(core_axis_name="c", subcore_axis_name="s")

    @functools.partial(
        pl.kernel,
        mesh=mesh,
        compiler_params=pltpu.CompilerParams(
            needs_layout_passes=False, use_tc_tiling_on_sc=True),
        out_type=jax.ShapeDtypeStruct((NW, 16), jnp.float32),
        scratch_types=(
            [pltpu.VMEM((ppw,), jnp.int32) for _ in range(2)]
            + [pltpu.VMEM((ppw,), jnp.float32)]
            + [pltpu.VMEM((C,), jnp.float32)]
            + [pltpu.VMEM((C, 2 * D), jnp.float32) for _ in range(2 * NBUF)]
            + [pltpu.VMEM((16,), jnp.float32)]
            + [pltpu.SemaphoreType.DMA for _ in range(2 * NBUF)]
        ),
    )
    def sc_loss(i0_h, i1_h, yse_h, w2_h, out_h, *refs):
        rawu, rawv, ysa, dots = refs[0:4]
        urows = refs[4:4 + NBUF]
        vrows = refs[4 + NBUF:4 + 2 * NBUF]
        accv = refs[4 + 2 * NBUF]
        semu = refs[5 + 2 * NBUF:5 + 2 * NBUF + NBUF]
        semv = refs[5 + 2 * NBUF + NBUF:5 + 2 * NBUF + 2 * NBUF]

        wid = lax.axis_index("s") * NC + lax.axis_index("c")
        tbase = wid * ppw
        lanes = lax.iota(jnp.int32, 16)

        pltpu.sync_copy(i0_h.at[pl.ds(tbase, ppw)], rawu)
        pltpu.sync_copy(i1_h.at[pl.ds(tbase, ppw)], rawv)
        pltpu.sync_copy(yse_h.at[pl.ds(tbase, ppw)], ysa)

        def fire(c, r):
            pltpu.async_copy(
                w2_h.at[rawu.at[pl.ds(c * C, C)]], urows[r], semu[r])
            pltpu.async_copy(
                w2_h.at[rawv.at[pl.ds(c * C, C)]], vrows[r], semv[r])

        def wait(c, r):
            pltpu.make_async_copy(
                w2_h.at[rawu.at[pl.ds(c * C, C)]], urows[r], semu[r]).wait()
            pltpu.make_async_copy(
                w2_h.at[rawv.at[pl.ds(c * C, C)]], vrows[r], semv[r]).wait()

        def compute(c, r, acc):
            def pair16(p16, _):
                dv = jnp.zeros((16,), jnp.float32)
                for q in range(16):
                    p = p16 * 16 + q
                    s = jnp.zeros((16,), jnp.float32)
                    for t in range(D // 16):
                        a = urows[r][p, pl.ds(t * 16, 16)]
                        b = vrows[r][p, pl.ds(t * 16, 16)]
                        s = s + a * b
                    dv = jnp.where(lanes == q, jnp.sum(s), dv)
                dots[pl.ds(p16 * 16, 16)] = dv
                return 0

            lax.fori_loop(0, C // 16, pair16, 0)

            def group(g, acc):
                avec = c * C + g * 16 + lanes
                dot = plsc.load_gather(dots, [g * 16 + lanes])
                z = jnp.where(tbase + avec < B, -dot, dot)
                t = jnp.exp(-jnp.abs(z))
                sp = jnp.maximum(z, 0.0) + _log_1to2(1.0 + t)
                yv = plsc.load_gather(ysa, [avec])
                alpha = _log_1to2(1.0 + _sqrt(yv)) + 1.0
                return acc + alpha * sp

            return lax.fori_loop(0, C // 16, group, acc)

        fire(0, 0)
        fire(1, 1)

        def step(k, acc):
            for r in range(NBUF):
                c = k * NBUF + r
                fire(c + 2, (r + 2) % NBUF)
                wait(c, r)
                acc = compute(c, r, acc)
            return acc

        acc = lax.fori_loop(0, nch // NBUF - 1, step,
                            jnp.zeros((16,), jnp.float32))
        c = nch - 3
        fire(nch - 1, (nch - 1) % NBUF)
        wait(c, c % NBUF)
        acc = compute(c, c % NBUF, acc)
        for c in range(nch - 2, nch):
            wait(c, c % NBUF)
            acc = compute(c, c % NBUF, acc)

        accv[...] = acc * scale
        pltpu.sync_copy(accv, out_h.at[wid])

    i0 = jnp.concatenate([pos_idxs[0], neg_idxs[0]])
    i1 = jnp.concatenate([pos_idxs[1], neg_idxs[1]])
    yse = jnp.concatenate([ys, jnp.zeros((NT,), jnp.float32)])
    w2 = _make_tc_transpose(V, D)(W.T)
    partials = sc_loss(i0, i1, yse, w2)
    return jnp.sum(partials)


# speedup vs baseline: 13.0951x; 1.0610x over previous
import functools

import jax
import jax.numpy as jnp
from jax import lax
from jax.experimental import pallas as pl
from jax.experimental.pallas import tpu as pltpu
from jax.experimental.pallas import tpu_sc as plsc

NC = 2
NS = 16
NW = NC * NS
C = 128
NBUF = 3


def _log_1to2(x):
    s = (x - 1.0) / (x + 1.0)
    s2 = s * s
    p = 1.0 / 9.0
    p = p * s2 + 1.0 / 7.0
    p = p * s2 + 1.0 / 5.0
    p = p * s2 + 1.0 / 3.0
    p = p * s2 + 1.0
    return (2.0 * s) * p


def _sqrt(x):
    i = lax.bitcast_convert_type(x, jnp.int32)
    y = lax.bitcast_convert_type(jnp.int32(0x5F3759DF) - (i >> 1), jnp.float32)
    for _ in range(3):
        y = y * (1.5 - 0.5 * x * y * y)
    return x * y


def _make_tc_transpose(V, D):
    BW = 16384
    VP = -(-V // BW) * BW
    grid = VP // BW

    return pl.pallas_call(
        _tc_transpose_body,
        grid=(grid,),
        in_specs=[pl.BlockSpec((D, BW), lambda b: (0, b))],
        out_specs=pl.BlockSpec((BW, 2 * D), lambda b: (b, 0)),
        out_shape=jax.ShapeDtypeStruct((VP, 2 * D), jnp.float32),
    )


def _tc_transpose_body(wt_ref, out_ref):
    x = wt_ref[...]
    out_ref[:, 0:x.shape[0]] = x.T


def kernel(pos_idxs, ys, neg_idxs, num_neg, W):
    B = pos_idxs.shape[1]
    NT = neg_idxs.shape[1]
    V, D = W.shape
    TOT = B + NT
    ppw = TOT // NW
    nch = ppw // C
    assert ppw % C == 0 and nch % NBUF == 0 and D == 64
    scale = 1.0 / float(TOT)

    mesh = plsc.VectorSubcoreMesh(core_axis_name="c", subcore_axis_name="s")

    @functools.partial(
        pl.kernel,
        mesh=mesh,
        compiler_params=pltpu.CompilerParams(
            needs_layout_passes=False, use_tc_tiling_on_sc=True),
        out_type=jax.ShapeDtypeStruct((NW, 16), jnp.float32),
        scratch_types=(
            [pltpu.VMEM((ppw,), jnp.int32) for _ in range(2)]
            + [pltpu.VMEM((ppw,), jnp.float32)]
            + [pltpu.VMEM((C,), jnp.float32)]
            + [pltpu.VMEM((C, 2 * D), jnp.float32) for _ in range(2 * NBUF)]
            + [pltpu.VMEM((16,), jnp.float32)]
            + [pltpu.SemaphoreType.DMA for _ in range(2 * NBUF)]
        ),
    )
    def sc_loss(i0_h, i1_h, yse_h, w2_h, out_h, *refs):
        rawu, rawv, ysa, dots = refs[0:4]
        urows = refs[4:4 + NBUF]
        vrows = refs[4 + NBUF:4 + 2 * NBUF]
        accv = refs[4 + 2 * NBUF]
        semu = refs[5 + 2 * NBUF:5 + 2 * NBUF + NBUF]
        semv = refs[5 + 2 * NBUF + NBUF:5 + 2 * NBUF + 2 * NBUF]

        wid = lax.axis_index("s") * NC + lax.axis_index("c")
        tbase = wid * ppw
        lanes = lax.iota(jnp.int32, 16)

        pltpu.sync_copy(i0_h.at[pl.ds(tbase, ppw)], rawu)
        pltpu.sync_copy(i1_h.at[pl.ds(tbase, ppw)], rawv)
        pltpu.sync_copy(yse_h.at[pl.ds(tbase, ppw)], ysa)

        def fire(c, r):
            pltpu.async_copy(
                w2_h.at[rawu.at[pl.ds(c * C, C)]], urows[r], semu[r])
            pltpu.async_copy(
                w2_h.at[rawv.at[pl.ds(c * C, C)]], vrows[r], semv[r])

        def wait(c, r):
            pltpu.make_async_copy(
                w2_h.at[rawu.at[pl.ds(c * C, C)]], urows[r], semu[r]).wait()
            pltpu.make_async_copy(
                w2_h.at[rawv.at[pl.ds(c * C, C)]], vrows[r], semv[r]).wait()

        def compute(c, r, acc):
            def pair16(p16, _):
                dv = jnp.zeros((16,), jnp.float32)
                for q in range(16):
                    p = p16 * 16 + q
                    s = jnp.zeros((16,), jnp.float32)
                    for t in range(D // 16):
                        a = urows[r][p, pl.ds(t * 16, 16)]
                        b = vrows[r][p, pl.ds(t * 16, 16)]
                        s = s + a * b
                    dv = jnp.where(lanes == q, jnp.sum(s), dv)
                dots[pl.ds(p16 * 16, 16)] = dv
                return 0

            lax.fori_loop(0, C // 16, pair16, 0)

            def group(g, acc):
                avec = c * C + g * 16 + lanes
                dot = plsc.load_gather(dots, [g * 16 + lanes])
                z = jnp.where(tbase + avec < B, -dot, dot)
                t = jnp.exp(-jnp.abs(z))
                sp = jnp.maximum(z, 0.0) + _log_1to2(1.0 + t)
                yv = plsc.load_gather(ysa, [avec])
                alpha = _log_1to2(1.0 + _sqrt(yv)) + 1.0
                return acc + alpha * sp

            return lax.fori_loop(0, C // 16, group, acc)

        fire(0, 0)
        fire(1, 1)

        def step(k, acc):
            for r in range(NBUF):
                c = k * NBUF + r
                fire(c + 2, (r + 2) % NBUF)
                wait(c, r)
                acc = compute(c, r, acc)
            return acc

        acc = lax.fori_loop(0, nch // NBUF - 1, step,
                            jnp.zeros((16,), jnp.float32))
        c = nch - 3
        fire(nch - 1, (nch - 1) % NBUF)
        wait(c, c % NBUF)
        acc = compute(c, c % NBUF, acc)
        for c in range(nch - 2, nch):
            wait(c, c % NBUF)
            acc = compute(c, c % NBUF, acc)

        accv[...] = acc * scale
        pltpu.sync_copy(accv, out_h.at[wid])

    i0 = jnp.concatenate([pos_idxs[0], neg_idxs[0]])
    i1 = jnp.concatenate([pos_idxs[1], neg_idxs[1]])
    yse = jnp.concatenate([ys, jnp.zeros((NT,), jnp.float32)])
    w2 = _make_tc_transpose(V, D)(W.T)
    partials = sc_loss(i0, i1, yse, w2)
    return jnp.sum(partials)
